# baseline (speedup 1.0000x reference)
import jax
import jax.numpy as jnp
from jax.experimental import pallas as pl
from jax.experimental.pallas import tpu as pltpu

IN_N = 2048
N_CHUNKS = 8


def _dma_body(x_ref, o_ref, sems):
    m = x_ref.shape[0]
    chunk = m // N_CHUNKS
    copies = [
        pltpu.make_async_copy(
            x_ref.at[pl.ds(i * chunk, chunk), :],
            o_ref.at[pl.ds(i * chunk, chunk), :],
            sems.at[i],
        )
        for i in range(N_CHUNKS)
    ]
    for c in copies:
        c.start()
    for c in copies:
        c.wait()


def kernel(input):
    x = input.reshape(-1, IN_N)
    return pl.pallas_call(
        _dma_body,
        in_specs=[pl.BlockSpec(memory_space=pl.ANY)],
        out_specs=pl.BlockSpec(memory_space=pl.ANY),
        out_shape=jax.ShapeDtypeStruct(x.shape, x.dtype),
        scratch_shapes=[pltpu.SemaphoreType.DMA((N_CHUNKS,))],
    )(x)


# speedup vs baseline: 49.1349x; 49.1349x over previous
import jax
import jax.numpy as jnp
from jax.experimental import pallas as pl
from jax.experimental.pallas import tpu as pltpu

IN_N = 2048
BLOCK_M = 1024


def _copy_body(x_ref, o_ref):
    o_ref[...] = x_ref[...]


def kernel(input):
    x = input.reshape(-1, IN_N)
    m, n = x.shape
    return pl.pallas_call(
        _copy_body,
        grid=(m // BLOCK_M,),
        in_specs=[pl.BlockSpec((BLOCK_M, n), lambda i: (i, 0))],
        out_specs=pl.BlockSpec((BLOCK_M, n), lambda i: (i, 0)),
        out_shape=jax.ShapeDtypeStruct((m, n), x.dtype),
    )(x)
